# baseline (device time: 9379 ns/iter reference)
import jax
import jax.numpy as jnp
from jax import lax
from jax.experimental import pallas as pl
from jax.experimental.pallas import tpu as pltpu

N_DEV = 8


def _hs_cumprod(x):
    m, n = x.shape
    d = 1
    while d < m:
        x = x * jnp.concatenate(
            [jnp.ones((d, n), x.dtype), x[: m - d, :]], axis=0
        )
        d *= 2
    return x


def _tree_prod(x):
    while x.shape[0] > 1:
        h = x.shape[0] // 2
        x = x[:h, :] * x[h:, :]
    return x


def kernel(x):
    m, n = x.shape

    def body(
        x_ref,
        out_ref,
        tot_ref,
        recv_buf,
        excl_ref,
        credit_sems,
        send_sems,
        recv_sems,
    ):
        my = lax.axis_index("i")
        f32 = jnp.float32
        excl_ref[0, :] = jnp.ones((n,), f32)

        for k in range(N_DEV):

            @pl.when(k < my)
            def _(k=k):
                pl.semaphore_signal(
                    credit_sems.at[my],
                    inc=1,
                    device_id=(k,),
                    device_id_type=pl.DeviceIdType.MESH,
                )

        xv = x_ref[...]
        tot_ref[0:1, :] = _tree_prod(xv)

        copies = [None] * N_DEV
        for j in reversed(range(N_DEV)):
            c = pltpu.make_async_remote_copy(
                src_ref=tot_ref,
                dst_ref=recv_buf.at[pl.ds(my, 1), :],
                send_sem=send_sems.at[j],
                recv_sem=recv_sems.at[my],
                device_id=(j,),
                device_id_type=pl.DeviceIdType.MESH,
            )
            copies[j] = c

            @pl.when(j > my)
            def _(c=c, j=j):
                pl.semaphore_wait(credit_sems.at[j], 1)
                c.start()

        out_ref[...] = _hs_cumprod(xv)

        for j in range(N_DEV):
            rc = pltpu.make_async_remote_copy(
                src_ref=tot_ref,
                dst_ref=recv_buf.at[pl.ds(j, 1), :],
                send_sem=send_sems.at[j],
                recv_sem=recv_sems.at[j],
                device_id=(j,),
                device_id_type=pl.DeviceIdType.MESH,
            )

            @pl.when(j < my)
            def _(rc=rc, j=j):
                rc.wait_recv()
                excl_ref[0, :] = excl_ref[0, :] * recv_buf[j, :]

        for j in range(N_DEV):

            @pl.when(j > my)
            def _(c=copies[j]):
                c.wait_send()

        out_ref[...] = out_ref[...] * excl_ref[0:1, :]

    return pl.pallas_call(
        body,
        out_shape=jax.ShapeDtypeStruct((m, n), jnp.float32),
        in_specs=[pl.BlockSpec(memory_space=pltpu.VMEM)],
        out_specs=pl.BlockSpec(memory_space=pltpu.VMEM),
        scratch_shapes=[
            pltpu.VMEM((1, n), jnp.float32),
            pltpu.VMEM((N_DEV, n), jnp.float32),
            pltpu.VMEM((1, n), jnp.float32),
            pltpu.SemaphoreType.REGULAR((N_DEV,)),
            pltpu.SemaphoreType.DMA((N_DEV,)),
            pltpu.SemaphoreType.DMA((N_DEV,)),
        ],
        compiler_params=pltpu.CompilerParams(skip_device_barrier=True),
    )(x)


# device time: 9374 ns/iter; 1.0005x vs baseline; 1.0005x over previous
import jax
import jax.numpy as jnp
from jax import lax
from jax.experimental import pallas as pl
from jax.experimental.pallas import tpu as pltpu

N_DEV = 8


def _hs_cumprod(x):
    m, n = x.shape
    d = 1
    while d < m:
        x = x * jnp.concatenate(
            [jnp.ones((d, n), x.dtype), x[: m - d, :]], axis=0
        )
        d *= 2
    return x


def _tree_prod(x):
    while x.shape[0] > 1:
        h = x.shape[0] // 2
        x = x[:h, :] * x[h:, :]
    return x


def kernel(x):
    m, n = x.shape

    def body(
        x_ref, out_ref, tot_ref, recv_buf, credit_sems, send_sems, recv_sems
    ):
        my = lax.axis_index("i")
        f32 = jnp.float32

        for k in range(N_DEV):

            @pl.when(k < my)
            def _(k=k):
                pl.semaphore_signal(
                    credit_sems.at[my],
                    inc=1,
                    device_id=(k,),
                    device_id_type=pl.DeviceIdType.MESH,
                )

        xv = x_ref[...]

        @pl.when(my < N_DEV - 1)
        def _():
            tot_ref[0:1, :] = _tree_prod(xv)

        copies = [None] * N_DEV
        for j in reversed(range(N_DEV)):
            c = pltpu.make_async_remote_copy(
                src_ref=tot_ref,
                dst_ref=recv_buf.at[pl.ds(my, 1), :],
                send_sem=send_sems.at[j],
                recv_sem=recv_sems.at[my],
                device_id=(j,),
                device_id_type=pl.DeviceIdType.MESH,
            )
            copies[j] = c

            @pl.when(j > my)
            def _(c=c, j=j):
                pl.semaphore_wait(credit_sems.at[j], 1)
                c.start()

        out_ref[...] = _hs_cumprod(xv)

        for j in reversed(range(N_DEV)):
            rc = pltpu.make_async_remote_copy(
                src_ref=tot_ref,
                dst_ref=recv_buf.at[pl.ds(j, 1), :],
                send_sem=send_sems.at[j],
                recv_sem=recv_sems.at[j],
                device_id=(j,),
                device_id_type=pl.DeviceIdType.MESH,
            )

            @pl.when(j < my)
            def _(rc=rc, j=j):
                rc.wait_recv()

        for j in range(N_DEV):

            @pl.when(j > my)
            def _(c=copies[j]):
                c.wait_send()

        r8 = recv_buf[...]
        mask = lax.broadcasted_iota(jnp.int32, (N_DEV, 1), 0) < my
        excl = _tree_prod(jnp.where(mask, r8, jnp.ones((N_DEV, n), f32)))
        out_ref[...] = out_ref[...] * excl

    return pl.pallas_call(
        body,
        out_shape=jax.ShapeDtypeStruct((m, n), jnp.float32),
        in_specs=[pl.BlockSpec(memory_space=pltpu.VMEM)],
        out_specs=pl.BlockSpec(memory_space=pltpu.VMEM),
        scratch_shapes=[
            pltpu.VMEM((1, n), jnp.float32),
            pltpu.VMEM((N_DEV, n), jnp.float32),
            pltpu.SemaphoreType.REGULAR((N_DEV,)),
            pltpu.SemaphoreType.DMA((N_DEV,)),
            pltpu.SemaphoreType.DMA((N_DEV,)),
        ],
        compiler_params=pltpu.CompilerParams(skip_device_barrier=True),
    )(x)
